# baseline (speedup 1.0000x reference)
import functools

import jax
import jax.numpy as jnp
from jax import lax
from jax.experimental import pallas as pl
from jax.experimental.pallas import tpu as pltpu
from jax.experimental.pallas import tpu_sc as plsc

_NUM_BUCKETS = 400
_RES = 20
_HALF_LIFE = 12.0
_TAIL_FRAC = 0.05
_MEAN_CPU = 0.5
_B = 8
_T = 160000
_W = _T // _RES
_CHUNKS = 4
_WARM = 256
_RESCALE = 32

_L = (_W - _WARM) // _CHUNKS
_L0 = _L + _WARM

_DECAY = 2.0 ** (-1.0 / _HALF_LIFE)
_INV_DECAY = 2.0 ** (1.0 / _HALF_LIFE)
_RESCALE_MUL = _DECAY ** _RESCALE

_IN_ELEMS = _L0 * _RES
_NGROUPS = _NUM_BUCKETS // 16


def _take16(x, idx):
    return lax.gather(
        x, idx[:, None],
        lax.GatherDimensionNumbers(offset_dims=(), collapsed_slice_dims=(0,),
                                   start_index_map=(0,)),
        (1,), mode=lax.GatherScatterMode.PROMISE_IN_BOUNDS)


def _suffix_cumsum(x):
    return lax.rev(jnp.cumsum(lax.rev(x, (0,)), axis=0), (0,))


def _sc_body(ts_hbm, buckets_hbm, out_hbm,
             in_v, rec_v, hist_a, gsum_a, buckets_v):
    c = lax.axis_index("c")
    s = lax.axis_index("s")
    wid = c * 16 + s
    series = wid // _CHUNKS
    t = wid % _CHUNKS

    is_first = t == 0
    warm = jnp.where(is_first, 0, _WARM)
    nwin = jnp.where(is_first, _L0, _L)
    start = jnp.where(is_first, 0, _L0 + (t - 1) * _L)
    in_off = series * _T + (start - warm) * _RES

    pltpu.sync_copy(ts_hbm.at[pl.ds(in_off, _IN_ELEMS)],
                    in_v.at[pl.ds(0, _IN_ELEMS)])
    pltpu.sync_copy(buckets_hbm, buckets_v)

    iota = lax.iota(jnp.int32, 16)
    zero16 = jnp.zeros((16,), jnp.float32)
    zero_i = jnp.zeros((16,), jnp.int32)
    mask4 = iota < 4
    lane0 = iota == 0

    for q in range(_NGROUPS + 1):
        hist_a[pl.ds(16 * q, 16)] = zero16
    gsum_a[pl.ds(0, 16)] = zero16

    def scatter_window(hist_v, gsum_v, idx0, idx1, wv):
        plsc.addupdate_scatter(hist_v, [idx0], wv)
        plsc.addupdate_scatter(hist_v, [idx1], wv, mask=mask4)
        plsc.addupdate_scatter(gsum_v, [idx0 >> 5], wv)
        plsc.addupdate_scatter(gsum_v, [idx1 >> 5], wv, mask=mask4)

    def load_idx(off):
        g0 = plsc.load_gather(in_v, [iota + off])
        g1 = plsc.load_gather(in_v, [iota + (off + 16)])
        g2 = plsc.load_gather(in_v, [iota + (off + 20)])
        g3 = plsc.load_gather(in_v, [iota + (off + 36)])
        return ((g0 * jnp.float32(_NUM_BUCKETS)).astype(jnp.int32),
                (g1 * jnp.float32(_NUM_BUCKETS)).astype(jnp.int32),
                (g2 * jnp.float32(_NUM_BUCKETS)).astype(jnp.int32),
                (g3 * jnp.float32(_NUM_BUCKETS)).astype(jnp.int32))

    def rescale_state():
        m = jnp.float32(_RESCALE_MUL)
        for q in range(_NGROUPS):
            hist_a[pl.ds(16 * q, 16)] = hist_a[pl.ds(16 * q, 16)] * m
        gsum_a[pl.ds(0, 16)] = gsum_a[pl.ds(0, 16)] * m

    def walk(j, tot, hist_v, gsum_v):
        tau = jnp.float32(_TAIL_FRAC) * tot
        s_g = _suffix_cumsum(gsum_v[pl.ds(0, 16)])
        n = plsc.all_reduce_population_count(s_g <= tau)
        above = _take16(s_g, 16 - n)
        hbase = (15 - n) * 32
        hv_a = plsc.load_gather(hist_v, [iota + hbase])
        hv_b = plsc.load_gather(hist_v, [iota + (hbase + 16)])
        cs_b = jnp.cumsum(lax.rev(hv_b, (0,)))
        cs_a = jnp.cumsum(lax.rev(hv_a, (0,)))
        s_b = lax.rev(cs_b, (0,)) + above
        s_a = lax.rev(cs_a, (0,)) + _take16(s_b, zero_i)
        nh_a = plsc.all_reduce_population_count(s_a <= tau)
        nh_b = plsc.all_reduce_population_count(s_b <= tau)
        pidx = hbase + 31 - nh_a - nh_b
        rv = (plsc.load_gather(buckets_v, [pidx])
              * jnp.float32(_MEAN_CPU))
        plsc.store_scatter(rec_v, [jnp.full((16,), j, jnp.int32)],
                           rv, mask=lane0)

    def step_scalars(w, tot, do_rescale):
        w1 = w * jnp.float32(_INV_DECAY)
        tot1 = tot + jnp.float32(_RES) * w + jnp.float32(_RES) * w1
        w2 = jnp.where(do_rescale, jnp.float32(1.0),
                       w1 * jnp.float32(_INV_DECAY))
        tot2 = jnp.where(do_rescale, tot1 * jnp.float32(_RESCALE_MUL), tot1)
        return w1, tot1, w2, tot2

    def warm_body(p, carry):
        w, tot = carry
        idx0, idx1, idx2, idx3 = load_idx(p * (2 * _RES))
        do_rescale = ((p * 2 + 1) & (_RESCALE - 1)) == (_RESCALE - 1)
        w1, tot1, w2, tot2 = step_scalars(w, tot, do_rescale)
        scatter_window(hist_a, gsum_a, idx0, idx1,
                       jnp.full((16,), w, jnp.float32))
        scatter_window(hist_a, gsum_a, idx2, idx3,
                       jnp.full((16,), w1, jnp.float32))

        @pl.when(do_rescale)
        def _rescale():
            rescale_state()

        return w2, tot2

    carry0 = lax.fori_loop(0, warm // 2, warm_body,
                           (jnp.float32(1.0), jnp.float32(0.0)))

    def pair_body(p, carry):
        w, tot = carry
        jr = p * 2
        idx0, idx1, idx2, idx3 = load_idx((warm + jr) * _RES)
        do_rescale = ((jr + 1) & (_RESCALE - 1)) == (_RESCALE - 1)
        w1, tot1, w2, tot2 = step_scalars(w, tot, do_rescale)
        tot0 = tot + jnp.float32(_RES) * w

        scatter_window(hist_a, gsum_a, idx0, idx1,
                       jnp.full((16,), w, jnp.float32))
        walk(jr, tot0, hist_a, gsum_a)
        scatter_window(hist_a, gsum_a, idx2, idx3,
                       jnp.full((16,), w1, jnp.float32))
        walk(jr + 1, tot1, hist_a, gsum_a)

        @pl.when(do_rescale)
        def _rescale():
            rescale_state()

        return w2, tot2

    lax.fori_loop(0, nwin // 2, pair_body, carry0)

    out_off = series * _W + start

    @pl.when(is_first)
    def _store_first():
        pltpu.sync_copy(rec_v.at[pl.ds(0, _L0)],
                        out_hbm.at[pl.ds(out_off, _L0)])

    @pl.when(jnp.logical_not(is_first))
    def _store_rest():
        pltpu.sync_copy(rec_v.at[pl.ds(0, _L)],
                        out_hbm.at[pl.ds(out_off, _L)])


_sc_kernel = pl.kernel(
    _sc_body,
    out_type=jax.ShapeDtypeStruct((_B * _W,), jnp.float32),
    mesh=plsc.VectorSubcoreMesh(core_axis_name="c", subcore_axis_name="s",
                                num_cores=2, num_subcores=16),
    compiler_params=pltpu.CompilerParams(needs_layout_passes=False),
    scratch_types=[
        pltpu.VMEM((_IN_ELEMS + 16,), jnp.float32),
        pltpu.VMEM((_L0,), jnp.float32),
        pltpu.VMEM((_NUM_BUCKETS + 16,), jnp.float32),
        pltpu.VMEM((16,), jnp.float32),
        pltpu.VMEM((_NUM_BUCKETS,), jnp.float32),
    ],
)


def kernel(time_series_list, cpu_buckets, cpu_bins):
    del cpu_bins
    rec = _sc_kernel(time_series_list.reshape(-1), cpu_buckets)
    return jnp.repeat(rec.reshape(_B, _W), _RES, axis=1)


# speedup vs baseline: 1.2577x; 1.2577x over previous
import functools

import jax
import jax.numpy as jnp
from jax import lax
from jax.experimental import pallas as pl
from jax.experimental.pallas import tpu as pltpu
from jax.experimental.pallas import tpu_sc as plsc

_NUM_BUCKETS = 400
_RES = 20
_HALF_LIFE = 12.0
_TAIL_FRAC = 0.05
_MEAN_CPU = 0.5
_B = 8
_T = 160000
_W = _T // _RES
_CHUNKS = 4
_WARM = 256
_RESCALE = 32

_L = (_W - _WARM) // _CHUNKS
_L0 = _L + _WARM

_DECAY = 2.0 ** (-1.0 / _HALF_LIFE)
_INV_DECAY = 2.0 ** (1.0 / _HALF_LIFE)
_RESCALE_MUL = _DECAY ** _RESCALE

_IN_ELEMS = _L0 * _RES
_NGROUPS = _NUM_BUCKETS // 16


def _take16(x, idx):
    return lax.gather(
        x, idx[:, None],
        lax.GatherDimensionNumbers(offset_dims=(), collapsed_slice_dims=(0,),
                                   start_index_map=(0,)),
        (1,), mode=lax.GatherScatterMode.PROMISE_IN_BOUNDS)


def _suffix_cumsum(x):
    return lax.rev(jnp.cumsum(lax.rev(x, (0,)), axis=0), (0,))


def _sc_body(ts_hbm, buckets_hbm, out_hbm,
             in_v, out_v, rec_v, hist_a, gsum_a, buckets_v):
    c = lax.axis_index("c")
    s = lax.axis_index("s")
    wid = c * 16 + s
    series = wid // _CHUNKS
    t = wid % _CHUNKS

    is_first = t == 0
    warm = jnp.where(is_first, 0, _WARM)
    nwin = jnp.where(is_first, _L0, _L)
    start = jnp.where(is_first, 0, _L0 + (t - 1) * _L)
    in_off = series * _T + (start - warm) * _RES

    pltpu.sync_copy(ts_hbm.at[pl.ds(in_off, _IN_ELEMS)],
                    in_v.at[pl.ds(0, _IN_ELEMS)])
    pltpu.sync_copy(buckets_hbm, buckets_v)

    iota = lax.iota(jnp.int32, 16)
    zero16 = jnp.zeros((16,), jnp.float32)
    zero_i = jnp.zeros((16,), jnp.int32)
    mask4 = iota < 4
    lane0 = iota == 0

    for q in range(_NGROUPS + 1):
        hist_a[pl.ds(16 * q, 16)] = zero16
    gsum_a[pl.ds(0, 16)] = zero16

    def scatter_window(hist_v, gsum_v, idx0, idx1, wv):
        plsc.addupdate_scatter(hist_v, [idx0], wv)
        plsc.addupdate_scatter(hist_v, [idx1], wv, mask=mask4)
        plsc.addupdate_scatter(gsum_v, [idx0 >> 5], wv)
        plsc.addupdate_scatter(gsum_v, [idx1 >> 5], wv, mask=mask4)

    def load_idx(off):
        g0 = plsc.load_gather(in_v, [iota + off])
        g1 = plsc.load_gather(in_v, [iota + (off + 16)])
        g2 = plsc.load_gather(in_v, [iota + (off + 20)])
        g3 = plsc.load_gather(in_v, [iota + (off + 36)])
        return ((g0 * jnp.float32(_NUM_BUCKETS)).astype(jnp.int32),
                (g1 * jnp.float32(_NUM_BUCKETS)).astype(jnp.int32),
                (g2 * jnp.float32(_NUM_BUCKETS)).astype(jnp.int32),
                (g3 * jnp.float32(_NUM_BUCKETS)).astype(jnp.int32))

    def rescale_state():
        m = jnp.float32(_RESCALE_MUL)
        for q in range(_NGROUPS):
            hist_a[pl.ds(16 * q, 16)] = hist_a[pl.ds(16 * q, 16)] * m
        gsum_a[pl.ds(0, 16)] = gsum_a[pl.ds(0, 16)] * m

    def walk(j, tot, hist_v, gsum_v):
        tau = jnp.float32(_TAIL_FRAC) * tot
        s_g = _suffix_cumsum(gsum_v[pl.ds(0, 16)])
        n = plsc.all_reduce_population_count(s_g <= tau)
        above = _take16(s_g, 16 - n)
        hbase = (15 - n) * 32
        hv_a = plsc.load_gather(hist_v, [iota + hbase])
        hv_b = plsc.load_gather(hist_v, [iota + (hbase + 16)])
        cs_b = jnp.cumsum(lax.rev(hv_b, (0,)))
        cs_a = jnp.cumsum(lax.rev(hv_a, (0,)))
        s_b = lax.rev(cs_b, (0,)) + above
        s_a = lax.rev(cs_a, (0,)) + _take16(s_b, zero_i)
        nh_a = plsc.all_reduce_population_count(s_a <= tau)
        nh_b = plsc.all_reduce_population_count(s_b <= tau)
        pidx = hbase + 31 - nh_a - nh_b
        rv = (plsc.load_gather(buckets_v, [pidx])
              * jnp.float32(_MEAN_CPU))
        plsc.store_scatter(rec_v, [jnp.full((16,), j, jnp.int32)],
                           rv, mask=lane0)

    def step_scalars(w, tot, do_rescale):
        w1 = w * jnp.float32(_INV_DECAY)
        tot1 = tot + jnp.float32(_RES) * w + jnp.float32(_RES) * w1
        w2 = jnp.where(do_rescale, jnp.float32(1.0),
                       w1 * jnp.float32(_INV_DECAY))
        tot2 = jnp.where(do_rescale, tot1 * jnp.float32(_RESCALE_MUL), tot1)
        return w1, tot1, w2, tot2

    def warm_body(p, carry):
        w, tot = carry
        idx0, idx1, idx2, idx3 = load_idx(p * (2 * _RES))
        do_rescale = ((p * 2 + 1) & (_RESCALE - 1)) == (_RESCALE - 1)
        w1, tot1, w2, tot2 = step_scalars(w, tot, do_rescale)
        scatter_window(hist_a, gsum_a, idx0, idx1,
                       jnp.full((16,), w, jnp.float32))
        scatter_window(hist_a, gsum_a, idx2, idx3,
                       jnp.full((16,), w1, jnp.float32))

        @pl.when(do_rescale)
        def _rescale():
            rescale_state()

        return w2, tot2

    carry0 = lax.fori_loop(0, warm // 2, warm_body,
                           (jnp.float32(1.0), jnp.float32(0.0)))

    def pair_body(p, carry):
        w, tot = carry
        jr = p * 2
        idx0, idx1, idx2, idx3 = load_idx((warm + jr) * _RES)
        do_rescale = ((jr + 1) & (_RESCALE - 1)) == (_RESCALE - 1)
        w1, tot1, w2, tot2 = step_scalars(w, tot, do_rescale)
        tot0 = tot + jnp.float32(_RES) * w

        scatter_window(hist_a, gsum_a, idx0, idx1,
                       jnp.full((16,), w, jnp.float32))
        walk(jr, tot0, hist_a, gsum_a)
        scatter_window(hist_a, gsum_a, idx2, idx3,
                       jnp.full((16,), w1, jnp.float32))
        walk(jr + 1, tot1, hist_a, gsum_a)

        @pl.when(do_rescale)
        def _rescale():
            rescale_state()

        return w2, tot2

    lax.fori_loop(0, nwin // 2, pair_body, carry0)

    maps = [(iota + 16 * q) // _RES for q in range(_RES)]

    def expand_body(g, dummy):
        for q in range(_RES):
            v = plsc.load_gather(rec_v, [maps[q] + g * 16])
            plsc.store_scatter(out_v, [iota + (g * 320 + q * 16)], v)
        return dummy

    lax.fori_loop(0, nwin // 16, expand_body, jnp.int32(0))

    out_off = series * _T + start * _RES

    @pl.when(is_first)
    def _store_first():
        pltpu.sync_copy(out_v.at[pl.ds(0, _L0 * _RES)],
                        out_hbm.at[pl.ds(out_off, _L0 * _RES)])

    @pl.when(jnp.logical_not(is_first))
    def _store_rest():
        pltpu.sync_copy(out_v.at[pl.ds(0, _L * _RES)],
                        out_hbm.at[pl.ds(out_off, _L * _RES)])


_sc_kernel = pl.kernel(
    _sc_body,
    out_type=jax.ShapeDtypeStruct((_B * _T,), jnp.float32),
    mesh=plsc.VectorSubcoreMesh(core_axis_name="c", subcore_axis_name="s",
                                num_cores=2, num_subcores=16),
    compiler_params=pltpu.CompilerParams(needs_layout_passes=False),
    scratch_types=[
        pltpu.VMEM((_IN_ELEMS + 16,), jnp.float32),
        pltpu.VMEM((_L0 * _RES,), jnp.float32),
        pltpu.VMEM((_L0,), jnp.float32),
        pltpu.VMEM((_NUM_BUCKETS + 16,), jnp.float32),
        pltpu.VMEM((16,), jnp.float32),
        pltpu.VMEM((_NUM_BUCKETS,), jnp.float32),
    ],
)


def kernel(time_series_list, cpu_buckets, cpu_bins):
    del cpu_bins
    flat = _sc_kernel(time_series_list.reshape(-1), cpu_buckets)
    return flat.reshape(_B, _T)
